# baseline (speedup 1.0000x reference)
import functools

import jax
import jax.numpy as jnp
from jax import lax
from jax.experimental import pallas as pl
from jax.experimental.pallas import tpu as pltpu
from jax.experimental.pallas import tpu_sc as plsc

T = 2048
D = 2048
E = 16
K = 8
F = 768

BT_R = 512
EG = 2
BTB = 1024

NWA = 16
TPW = T // NWA


def _logits_body(x_ref, wr_ref, lg_ref, xb_ref):
    xb_ref[...] = x_ref[...].astype(jnp.bfloat16)
    logits = jnp.dot(x_ref[...], wr_ref[...],
                     preferred_element_type=jnp.float32)
    lg_ref[...] = logits.T


def _sc_router_body(lg_hbm, comb_hbm, lg_v, cb_v):
    wid = lax.axis_index("s") * 2 + lax.axis_index("c")

    @pl.when(wid < NWA)
    def _():
        _sc_router_work(wid, lg_hbm, comb_hbm, lg_v, cb_v)


def _sc_router_work(wid, lg_hbm, comb_hbm, lg_v, cb_v):
    base = wid * TPW
    pltpu.sync_copy(lg_hbm.at[:, pl.ds(base, TPW)], lg_v)

    one = jnp.ones((16,), jnp.float32)
    zero = jnp.zeros((16,), jnp.float32)
    neginf = jnp.full((16,), -jnp.inf, jnp.float32)
    for gidx in range(TPW // 16):
        sl = pl.ds(gidx * 16, 16)
        le = [lg_v[e, sl] for e in range(E)]
        m = le[0]
        for e in range(1, E):
            m = jnp.maximum(m, le[e])
        pe = [jnp.exp(le[e] - m) for e in range(E)]
        s = pe[0]
        for e in range(1, E):
            s = s + pe[e]
        pe = [pe[e] / s for e in range(E)]
        pw = list(pe)
        sel = [zero] * E
        for _ in range(K):
            mk = pw[0]
            for e in range(1, E):
                mk = jnp.maximum(mk, pw[e])
            rem = one
            for e in range(E):
                hit = jnp.where(pw[e] == mk, rem, zero)
                rem = rem - hit
                sel[e] = sel[e] + hit
                pw[e] = jnp.where(hit > zero, neginf, pw[e])
        ws = [pe[e] * sel[e] for e in range(E)]
        t = ws[0]
        for e in range(1, E):
            t = t + ws[e]
        for e in range(E):
            cb_v[e, sl] = ws[e] / t

    pltpu.sync_copy(cb_v, comb_hbm.at[:, pl.ds(base, TPW)])


_sc_router = functools.partial(
    pl.kernel,
    mesh=plsc.VectorSubcoreMesh(core_axis_name="c", subcore_axis_name="s"),
    out_type=jax.ShapeDtypeStruct((E, T), jnp.float32),
    scratch_types=[
        pltpu.VMEM((E, TPW), jnp.float32),
        pltpu.VMEM((E, TPW), jnp.float32),
    ],
)(_sc_router_body)


def _transpose_body(ct_ref, c_ref):
    c_ref[...] = ct_ref[...].T


def _gateup_body(x_ref, wg_ref, wu_ref, comb_ref, h_ref):
    e = pl.program_id(0)
    xb = x_ref[...]
    g = jnp.dot(xb, wg_ref[0].astype(jnp.bfloat16),
                preferred_element_type=jnp.float32)
    u = jnp.dot(xb, wu_ref[0].astype(jnp.bfloat16),
                preferred_element_type=jnp.float32)
    lane = jax.lax.broadcasted_iota(jnp.int32, (1, E), 1)
    w = jnp.sum(jnp.where(lane == e, comb_ref[...], 0.0), axis=1, keepdims=True)
    h = g * jax.nn.sigmoid(g) * u * w
    h_ref[...] = h.astype(jnp.bfloat16)


def _down_body(h_ref, wd_ref, out_ref):
    g = pl.program_id(1)
    y = jnp.dot(h_ref[...], wd_ref[...].astype(jnp.bfloat16),
                preferred_element_type=jnp.float32)

    @pl.when(g == 0)
    def _():
        out_ref[...] = y

    @pl.when(g > 0)
    def _():
        out_ref[...] += y


def kernel(x, W_router, w_gate, w_up, w_down):
    logits, xb = pl.pallas_call(
        _logits_body,
        grid=(T // BT_R,),
        in_specs=[
            pl.BlockSpec((BT_R, D), lambda t: (t, 0)),
            pl.BlockSpec((D, E), lambda t: (0, 0)),
        ],
        out_specs=[
            pl.BlockSpec((E, BT_R), lambda t: (0, t)),
            pl.BlockSpec((BT_R, D), lambda t: (t, 0)),
        ],
        out_shape=[
            jax.ShapeDtypeStruct((E, T), jnp.float32),
            jax.ShapeDtypeStruct((T, D), jnp.bfloat16),
        ],
    )(x, W_router)

    combine_t = _sc_router(logits)
    combine = pl.pallas_call(
        _transpose_body,
        grid=(1,),
        in_specs=[pl.BlockSpec((E, T), lambda i: (0, 0))],
        out_specs=pl.BlockSpec((T, E), lambda i: (0, 0)),
        out_shape=jax.ShapeDtypeStruct((T, E), jnp.float32),
    )(combine_t)

    wg = w_gate
    wu = w_up
    wd = w_down.reshape(E * F, D)

    h = pl.pallas_call(
        _gateup_body,
        grid=(E,),
        in_specs=[
            pl.BlockSpec((T, D), lambda e: (0, 0)),
            pl.BlockSpec((1, D, F), lambda e: (e, 0, 0)),
            pl.BlockSpec((1, D, F), lambda e: (e, 0, 0)),
            pl.BlockSpec((T, E), lambda e: (0, 0)),
        ],
        out_specs=pl.BlockSpec((T, F), lambda e: (0, e)),
        out_shape=jax.ShapeDtypeStruct((T, E * F), jnp.bfloat16),
    )(xb, wg, wu, combine)

    out = pl.pallas_call(
        _down_body,
        grid=(T // BTB, E // EG),
        in_specs=[
            pl.BlockSpec((BTB, EG * F), lambda t, g: (t, g)),
            pl.BlockSpec((EG * F, D), lambda t, g: (g, 0)),
        ],
        out_specs=pl.BlockSpec((BTB, D), lambda t, g: (t, 0)),
        out_shape=jax.ShapeDtypeStruct((T, D), jnp.float32),
    )(h, wd)
    return out


# speedup vs baseline: 1.0307x; 1.0307x over previous
import functools

import jax
import jax.numpy as jnp
from jax import lax
from jax.experimental import pallas as pl
from jax.experimental.pallas import tpu as pltpu
from jax.experimental.pallas import tpu_sc as plsc

T = 2048
D = 2048
E = 16
K = 8
F = 768

BT_R = 512
EG = 2
BTB = 1024

NWA = 16
TPW = T // NWA


def _logits_body(x_ref, wr_ref, lg_ref, xb_ref):
    xb_ref[...] = x_ref[...].astype(jnp.bfloat16)
    logits = jnp.dot(x_ref[...], wr_ref[...],
                     preferred_element_type=jnp.float32)
    lg_ref[...] = logits.T


def _sc_router_body(lg_hbm, comb_hbm, lg_v, cb_v):
    wid = lax.axis_index("s") * 2 + lax.axis_index("c")

    @pl.when(wid < NWA)
    def _():
        _sc_router_work(wid, lg_hbm, comb_hbm, lg_v, cb_v)


def _sc_router_work(wid, lg_hbm, comb_hbm, lg_v, cb_v):
    base = wid * TPW
    pltpu.sync_copy(lg_hbm.at[:, pl.ds(base, TPW)], lg_v)

    one = jnp.ones((16,), jnp.float32)
    zero = jnp.zeros((16,), jnp.float32)
    neginf = jnp.full((16,), -jnp.inf, jnp.float32)
    for gidx in range(TPW // 16):
        sl = pl.ds(gidx * 16, 16)
        le = [lg_v[e, sl] for e in range(E)]
        m = le[0]
        for e in range(1, E):
            m = jnp.maximum(m, le[e])
        pe = [jnp.exp(le[e] - m) for e in range(E)]
        s = pe[0]
        for e in range(1, E):
            s = s + pe[e]
        pe = [pe[e] / s for e in range(E)]
        pw = list(pe)
        sel = [zero] * E
        for _ in range(K):
            mk = pw[0]
            for e in range(1, E):
                mk = jnp.maximum(mk, pw[e])
            rem = one
            for e in range(E):
                hit = jnp.where(pw[e] == mk, rem, zero)
                rem = rem - hit
                sel[e] = sel[e] + hit
                pw[e] = jnp.where(hit > zero, neginf, pw[e])
        ws = [pe[e] * sel[e] for e in range(E)]
        t = ws[0]
        for e in range(1, E):
            t = t + ws[e]
        for e in range(E):
            cb_v[e, sl] = ws[e] / t

    pltpu.sync_copy(cb_v, comb_hbm.at[:, pl.ds(base, TPW)])


_sc_router = functools.partial(
    pl.kernel,
    mesh=plsc.VectorSubcoreMesh(core_axis_name="c", subcore_axis_name="s"),
    out_type=jax.ShapeDtypeStruct((E, T), jnp.float32),
    scratch_types=[
        pltpu.VMEM((E, TPW), jnp.float32),
        pltpu.VMEM((E, TPW), jnp.float32),
    ],
)(_sc_router_body)


def _transpose_body(ct_ref, c_ref):
    c_ref[...] = ct_ref[...].T


def _gateup_body(x_ref, wg_ref, wu_ref, h_ref):
    xb = x_ref[...]
    g = jnp.dot(xb, wg_ref[0].astype(jnp.bfloat16),
                preferred_element_type=jnp.float32)
    u = jnp.dot(xb, wu_ref[0].astype(jnp.bfloat16),
                preferred_element_type=jnp.float32)
    h = g * jax.nn.sigmoid(g) * u
    h_ref[...] = h.astype(jnp.bfloat16)


def _down_body(h_ref, comb_ref, wd_ref, out_ref):
    g = pl.program_id(1)
    lane = jax.lax.broadcasted_iota(jnp.int32, (1, E), 1)
    comb = comb_ref[...]
    h = h_ref[...]
    parts = []
    for j in range(EG):
        w = jnp.sum(jnp.where(lane == g * EG + j, comb, 0.0),
                    axis=1, keepdims=True)
        parts.append((h[:, j * F:(j + 1) * F].astype(jnp.float32)
                      * w).astype(jnp.bfloat16))
    hb = jnp.concatenate(parts, axis=1)
    y = jnp.dot(hb, wd_ref[...].astype(jnp.bfloat16),
                preferred_element_type=jnp.float32)

    @pl.when(g == 0)
    def _():
        out_ref[...] = y

    @pl.when(g > 0)
    def _():
        out_ref[...] += y


def kernel(x, W_router, w_gate, w_up, w_down):
    logits, xb = pl.pallas_call(
        _logits_body,
        grid=(T // BT_R,),
        in_specs=[
            pl.BlockSpec((BT_R, D), lambda t: (t, 0)),
            pl.BlockSpec((D, E), lambda t: (0, 0)),
        ],
        out_specs=[
            pl.BlockSpec((E, BT_R), lambda t: (0, t)),
            pl.BlockSpec((BT_R, D), lambda t: (t, 0)),
        ],
        out_shape=[
            jax.ShapeDtypeStruct((E, T), jnp.float32),
            jax.ShapeDtypeStruct((T, D), jnp.bfloat16),
        ],
    )(x, W_router)

    combine_t = _sc_router(logits)
    combine = pl.pallas_call(
        _transpose_body,
        grid=(1,),
        in_specs=[pl.BlockSpec((E, T), lambda i: (0, 0))],
        out_specs=pl.BlockSpec((T, E), lambda i: (0, 0)),
        out_shape=jax.ShapeDtypeStruct((T, E), jnp.float32),
    )(combine_t)

    wg = w_gate
    wu = w_up
    wd = w_down.reshape(E * F, D)

    h = pl.pallas_call(
        _gateup_body,
        grid=(E,),
        in_specs=[
            pl.BlockSpec((T, D), lambda e: (0, 0)),
            pl.BlockSpec((1, D, F), lambda e: (e, 0, 0)),
            pl.BlockSpec((1, D, F), lambda e: (e, 0, 0)),
        ],
        out_specs=pl.BlockSpec((T, F), lambda e: (0, e)),
        out_shape=jax.ShapeDtypeStruct((T, E * F), jnp.bfloat16),
    )(xb, wg, wu)

    out = pl.pallas_call(
        _down_body,
        grid=(T // BTB, E // EG),
        in_specs=[
            pl.BlockSpec((BTB, EG * F), lambda t, g: (t, g)),
            pl.BlockSpec((BTB, E), lambda t, g: (t, 0)),
            pl.BlockSpec((EG * F, D), lambda t, g: (g, 0)),
        ],
        out_specs=pl.BlockSpec((BTB, D), lambda t, g: (t, 0)),
        out_shape=jax.ShapeDtypeStruct((T, D), jnp.float32),
    )(h, combine, wd)
    return out
